# baseline (speedup 1.0000x reference)
import functools

import jax
import jax.numpy as jnp
from jax import lax
from jax.experimental import pallas as pl
from jax.experimental.pallas import tpu as pltpu
from jax.experimental.pallas import tpu_sc as plsc

B = 4096
H = 200
V = 1000000
D = 64
NC, NS = 2, 16
NW = NC * NS
R = B * H
RPW = R // NW
NU = RPW // 128
NG = 2
NO = 2
SCALE = 8.0


@functools.lru_cache(maxsize=None)
def _make():
    mesh = plsc.VectorSubcoreMesh(core_axis_name="c", subcore_axis_name="s")

    @functools.partial(
        pl.kernel,
        mesh=mesh,
        out_type=jax.ShapeDtypeStruct((R, D), jnp.float32),
        scratch_types=[
            pltpu.VMEM((NU, 128), jnp.int32),
            pltpu.VMEM((NG, 128, 2 * D), jnp.float32),
            pltpu.VMEM((NG, 128, 16), jnp.float32),
            pltpu.VMEM((NO, 128, D), jnp.float32),
            pltpu.SemaphoreType.DMA((NG,)),
            pltpu.SemaphoreType.DMA((NG,)),
            pltpu.SemaphoreType.DMA((NO,)),
        ],
    )
    def k(ix_hbm, tab_hbm, p8_hbm, out_hbm, xb, gbuf, pbuf, obuf,
          gsem, psem, osem):
        wid = lax.axis_index("s") * NC + lax.axis_index("c")
        u0 = wid * NU

        pltpu.sync_copy(ix_hbm.at[pl.ds(u0, NU)], xb)

        def fire_gather(u, g):
            pltpu.async_copy(tab_hbm.at[xb.at[u]], gbuf.at[g], gsem.at[g])
            pltpu.async_copy(
                p8_hbm.at[pl.ds((u0 + u) * 128, 128)], pbuf.at[g], psem.at[g]
            )

        def wait_gather(u, g):
            pltpu.make_async_copy(
                tab_hbm.at[xb.at[u]], gbuf.at[g], gsem.at[g]
            ).wait()
            pltpu.make_async_copy(
                p8_hbm.at[pl.ds((u0 + u) * 128, 128)], pbuf.at[g], psem.at[g]
            ).wait()

        def out_slice(u):
            return out_hbm.at[pl.ds((u0 + u) * 128, 128)]

        def fire_out(u, o):
            pltpu.async_copy(obuf.at[o], out_slice(u), osem.at[o])

        def wait_out(u, o):
            pltpu.make_async_copy(obuf.at[o], out_slice(u), osem.at[o]).wait()

        def blend(g, o):
            def body(i, carry):
                p = pbuf[g, i]
                for c in range(D // 16):
                    lo = gbuf[g, i, pl.ds(c * 16, 16)]
                    hi = gbuf[g, i, pl.ds(D + c * 16, 16)]
                    obuf[o, i, pl.ds(c * 16, 16)] = (
                        lo * SCALE + (hi - lo) * p
                    )
                return carry

            lax.fori_loop(0, 128, body, 0)

        for j in range(NG):
            fire_gather(j, j)

        def unit_body(u, carry):
            g = lax.rem(u, NG)
            o = lax.rem(u, NO)
            wait_gather(u, g)

            @pl.when(u >= NO)
            def _wo():
                wait_out(u - NO, o)

            blend(g, o)
            fire_out(u, o)

            @pl.when(u + NG < NU)
            def _fg():
                fire_gather(u + NG, g)

            return carry

        lax.fori_loop(0, NU, unit_body, 0)

        for u in range(NU - NO, NU):
            wait_out(u, u % NO)

    return k


@jax.jit
def kernel(x, input_embedding):
    xf = x.reshape(R).astype(jnp.int32)
    ix = (xf >> 1).reshape(R // 128, 128)
    p8 = jnp.broadcast_to(
        ((xf & 1).astype(jnp.float32) * SCALE)[:, None], (R, 16)
    )
    tab2 = input_embedding.reshape(V // 2, 2 * D)
    outp = _make()(ix, tab2, p8)
    return outp.reshape(B, H, D)


# speedup vs baseline: 1.2112x; 1.2112x over previous
import functools

import jax
import jax.numpy as jnp
from jax import lax
from jax.experimental import pallas as pl
from jax.experimental.pallas import tpu as pltpu
from jax.experimental.pallas import tpu_sc as plsc

B = 4096
H = 200
V = 1000000
D = 64
NC, NS = 2, 16
NW = NC * NS
R = B * H
RPW = R // NW
NU = RPW // 128
NG = 4
NO = 2
SCALE = 8.0


@functools.lru_cache(maxsize=None)
def _make():
    mesh = plsc.VectorSubcoreMesh(core_axis_name="c", subcore_axis_name="s")

    @functools.partial(
        pl.kernel,
        mesh=mesh,
        out_type=jax.ShapeDtypeStruct((R, D), jnp.float32),
        scratch_types=[
            pltpu.VMEM((NG, 128), jnp.int32),
            pltpu.VMEM((NG, 128, 2 * D), jnp.float32),
            pltpu.VMEM((NG, 16, 128), jnp.float32),
            pltpu.VMEM((NO, 128, D), jnp.float32),
            pltpu.SemaphoreType.DMA((NG,)),
            pltpu.SemaphoreType.DMA((NG,)),
            pltpu.SemaphoreType.DMA((NG,)),
            pltpu.SemaphoreType.DMA((NO,)),
        ],
    )
    def k(ix_hbm, tab_hbm, p8_hbm, out_hbm, xb, gbuf, pbuf, obuf,
          isem, gsem, psem, osem):
        wid = lax.axis_index("s") * NC + lax.axis_index("c")
        u0 = wid * NU

        def fire_idx(u, g):
            pltpu.async_copy(ix_hbm.at[u0 + u], xb.at[g], isem.at[g])

        def wait_idx(u, g):
            pltpu.make_async_copy(
                ix_hbm.at[u0 + u], xb.at[g], isem.at[g]
            ).wait()

        def fire_gather(u, g):
            pltpu.async_copy(tab_hbm.at[xb.at[g]], gbuf.at[g], gsem.at[g])
            pltpu.async_copy(p8_hbm.at[u0 + u], pbuf.at[g], psem.at[g])

        def wait_gather(u, g):
            pltpu.make_async_copy(
                tab_hbm.at[xb.at[g]], gbuf.at[g], gsem.at[g]
            ).wait()
            pltpu.make_async_copy(
                p8_hbm.at[u0 + u], pbuf.at[g], psem.at[g]
            ).wait()

        def out_slice(u):
            return out_hbm.at[pl.ds((u0 + u) * 128, 128)]

        def fire_out(u, o):
            pltpu.async_copy(obuf.at[o], out_slice(u), osem.at[o])

        def wait_out(u, o):
            pltpu.make_async_copy(obuf.at[o], out_slice(u), osem.at[o]).wait()

        def blend(g, o):
            def body(j, carry):
                for m in range(8):
                    i = j * 8 + m
                    p = pbuf[g, j, pl.ds(m * 16, 16)]
                    for c in range(D // 16):
                        lo = gbuf[g, i, pl.ds(c * 16, 16)]
                        hi = gbuf[g, i, pl.ds(D + c * 16, 16)]
                        obuf[o, i, pl.ds(c * 16, 16)] = (
                            lo * SCALE + (hi - lo) * p
                        )
                return carry

            lax.fori_loop(0, 16, body, 0)

        for j in range(NG):
            fire_idx(j, j)
        for j in range(NG):
            wait_idx(j, j)
            fire_gather(j, j)

        def unit_body(u, carry):
            g = lax.rem(u, NG)
            o = lax.rem(u, NO)
            wait_gather(u, g)

            @pl.when(u + NG < NU)
            def _fi():
                fire_idx(u + NG, g)

            @pl.when(u >= NO)
            def _wo():
                wait_out(u - NO, o)

            blend(g, o)
            fire_out(u, o)

            @pl.when(u + NG < NU)
            def _fg():
                wait_idx(u + NG, g)
                fire_gather(u + NG, g)

            return carry

        lax.fori_loop(0, NU, unit_body, 0)

        for u in range(NU - NO, NU):
            wait_out(u, u % NO)

    return k


@jax.jit
def kernel(x, input_embedding):
    xf = x.reshape(R).astype(jnp.int32)
    ix = (xf >> 1).reshape(R // 128, 128)
    p8 = jnp.broadcast_to(
        ((xf & 1).astype(jnp.float32) * SCALE)[:, None], (R, 16)
    ).reshape(R // 128, 16, 128)
    tab2 = input_embedding.reshape(V // 2, 2 * D)
    outp = _make()(ix, tab2, p8)
    return outp.reshape(B, H, D)


# speedup vs baseline: 1.2132x; 1.0017x over previous
import functools

import jax
import jax.numpy as jnp
from jax import lax
from jax.experimental import pallas as pl
from jax.experimental.pallas import tpu as pltpu
from jax.experimental.pallas import tpu_sc as plsc

B = 4096
H = 200
V = 1000000
D = 64
NC, NS = 2, 16
NW = NC * NS
R = B * H
RPW = R // NW
NU = RPW // 128
NG = 5
NO = 2
SCALE = 8.0


@functools.lru_cache(maxsize=None)
def _make():
    mesh = plsc.VectorSubcoreMesh(core_axis_name="c", subcore_axis_name="s")

    @functools.partial(
        pl.kernel,
        mesh=mesh,
        out_type=jax.ShapeDtypeStruct((R, D), jnp.float32),
        scratch_types=[
            pltpu.VMEM((NG, 128), jnp.int32),
            pltpu.VMEM((NG, 128, 2 * D), jnp.float32),
            pltpu.VMEM((NG, 16, 128), jnp.float32),
            pltpu.VMEM((NO, 128, D), jnp.float32),
            pltpu.SemaphoreType.DMA((NG,)),
            pltpu.SemaphoreType.DMA((NG,)),
            pltpu.SemaphoreType.DMA((NG,)),
            pltpu.SemaphoreType.DMA((NO,)),
        ],
    )
    def k(ix_hbm, tab_hbm, p8_hbm, out_hbm, xb, gbuf, pbuf, obuf,
          isem, gsem, psem, osem):
        wid = lax.axis_index("s") * NC + lax.axis_index("c")
        u0 = wid * NU

        def fire_idx(u, g):
            pltpu.async_copy(ix_hbm.at[u0 + u], xb.at[g], isem.at[g])

        def wait_idx(u, g):
            pltpu.make_async_copy(
                ix_hbm.at[u0 + u], xb.at[g], isem.at[g]
            ).wait()

        def fire_gather(u, g):
            pltpu.async_copy(tab_hbm.at[xb.at[g]], gbuf.at[g], gsem.at[g])
            pltpu.async_copy(p8_hbm.at[u0 + u], pbuf.at[g], psem.at[g])

        def wait_gather(u, g):
            pltpu.make_async_copy(
                tab_hbm.at[xb.at[g]], gbuf.at[g], gsem.at[g]
            ).wait()
            pltpu.make_async_copy(
                p8_hbm.at[u0 + u], pbuf.at[g], psem.at[g]
            ).wait()

        def out_slice(u):
            return out_hbm.at[pl.ds((u0 + u) * 128, 128)]

        def fire_out(u, o):
            pltpu.async_copy(obuf.at[o], out_slice(u), osem.at[o])

        def wait_out(u, o):
            pltpu.make_async_copy(obuf.at[o], out_slice(u), osem.at[o]).wait()

        def blend(g, o):
            def body(j, carry):
                for m in range(8):
                    i = j * 8 + m
                    p = pbuf[g, j, pl.ds(m * 16, 16)]
                    for c in range(D // 16):
                        lo = gbuf[g, i, pl.ds(c * 16, 16)]
                        hi = gbuf[g, i, pl.ds(D + c * 16, 16)]
                        obuf[o, i, pl.ds(c * 16, 16)] = (
                            lo * SCALE + (hi - lo) * p
                        )
                return carry

            lax.fori_loop(0, 16, body, 0)

        for j in range(NG):
            fire_idx(j, j)
        for j in range(NG):
            wait_idx(j, j)
            fire_gather(j, j)

        def unit_body(u, carry):
            g = lax.rem(u, NG)
            o = lax.rem(u, NO)
            wait_gather(u, g)

            @pl.when(u + NG < NU)
            def _fi():
                fire_idx(u + NG, g)

            @pl.when(u >= NO)
            def _wo():
                wait_out(u - NO, o)

            blend(g, o)
            fire_out(u, o)

            @pl.when(u + NG < NU)
            def _fg():
                wait_idx(u + NG, g)
                fire_gather(u + NG, g)

            return carry

        lax.fori_loop(0, NU, unit_body, 0)

        for u in range(NU - NO, NU):
            wait_out(u, u % NO)

    return k


@jax.jit
def kernel(x, input_embedding):
    xf = x.reshape(R).astype(jnp.int32)
    ix = (xf >> 1).reshape(R // 128, 128)
    p8 = jnp.broadcast_to(
        ((xf & 1).astype(jnp.float32) * SCALE)[:, None], (R, 16)
    ).reshape(R // 128, 16, 128)
    tab2 = input_embedding.reshape(V // 2, 2 * D)
    outp = _make()(ix, tab2, p8)
    return outp.reshape(B, H, D)
